# baseline (speedup 1.0000x reference)
import functools

import jax
import jax.numpy as jnp
from jax import lax
from jax.experimental import pallas as pl
from jax.experimental.pallas import tpu as pltpu
from jax.experimental.pallas import tpu_sc as plsc

T = 4096
L = 16
NCHUNK = T // L
BIG = T + 10


def _sc_body(in_hbm, out_hbm, e_v, out_v, stage_v):
    b = lax.axis_index("s")
    f = lax.axis_index("c")

    pltpu.sync_copy(in_hbm.at[b, pl.ds(2 * f, 2), :], stage_v)

    lane = lax.iota(jnp.int32, L)
    _dnums = lax.GatherDimensionNumbers(
        offset_dims=(), collapsed_slice_dims=(0,), start_index_map=(0,)
    )
    _first = jnp.zeros((L, 1), jnp.int32)
    _last = jnp.full((L, 1), L - 1, jnp.int32)
    _shl = jnp.minimum(lane + 1, L - 1)[:, None]
    _shr = jnp.maximum(lane - 1, 0)[:, None]

    def _gat(v, idx):
        return lax.gather(
            v, idx, _dnums, (1,), mode=lax.GatherScatterMode.PROMISE_IN_BOUNDS
        )

    GRP = 8
    NGRP = NCHUNK // GRP

    @plsc.parallel_loop(
        0,
        NGRP,
        unroll=2,
        carry=(jnp.full((L,), -BIG, jnp.int32), jnp.zeros((L,), jnp.float32)),
    )
    def _bwd(g, carry):
        cmax, nf = carry
        gbase = (NGRP - 1 - g) * GRP * L
        nf_next = None
        for j in range(GRP - 1, -1, -1):
            base = gbase + j * L
            cur = stage_v[1, pl.ds(base, L)]
            if j == GRP - 1:
                nxt = jnp.where(lane == L - 1, nf, _gat(cur, _shl))
            else:
                nxt = stage_v[1, pl.ds(base + 1, L)]
            if j == 0:
                nf_next = _gat(cur, _first)
            on = nxt > cur
            negpos = jnp.where(on, -(base + 1) - lane, -BIG)
            cmax = jnp.maximum(
                jnp.flip(plsc.cummax(jnp.flip(negpos, 0)), 0), cmax
            )
            e_v[pl.ds(base, L)] = jnp.minimum(8 - cmax, T)
            cmax = _gat(cmax, _first)
        return cmax, nf_next

    @plsc.parallel_loop(
        0,
        NGRP,
        unroll=2,
        carry=(jnp.zeros((L,), jnp.int32), jnp.zeros((L,), jnp.float32)),
    )
    def _fwd(g, carry):
        mmax, pl_last = carry
        gbase = g * GRP * L
        pl_next = None
        for j in range(GRP):
            base = gbase + j * L
            pcur = stage_v[0, pl.ds(base, L)]
            if j == 0:
                pprev = jnp.where(lane == 0, pl_last, _gat(pcur, _shr))
            else:
                pprev = stage_v[0, pl.ds(base - 1, L)]
            if j == GRP - 1:
                pl_next = _gat(pcur, _last)
            e = jnp.where(pcur > pprev, e_v[pl.ds(base, L)], 0)
            mmax = jnp.maximum(plsc.cummax(e), mmax)
            out_v[pl.ds(base, L)] = jnp.where(mmax > base + lane, 1.0, 0.0)
            mmax = _gat(mmax, _last)
        return mmax, pl_next

    pltpu.sync_copy(out_v, out_hbm.at[b, f])


@jax.jit
def _run(x):
    mesh = plsc.VectorSubcoreMesh(core_axis_name="c", subcore_axis_name="s")
    f = pl.kernel(
        _sc_body,
        out_type=jax.ShapeDtypeStruct((16, 2, T), jnp.float32),
        mesh=mesh,
        scratch_types=[
            pltpu.VMEM((T,), jnp.int32),
            pltpu.VMEM((T,), jnp.float32),
            pltpu.VMEM((2, T), jnp.float32),
        ],
        compiler_params=pltpu.CompilerParams(
            needs_layout_passes=False, use_tc_tiling_on_sc=True
        ),
    )
    return f(x)


def kernel(gesture_labels):
    return _run(gesture_labels)


# speedup vs baseline: 1.2726x; 1.2726x over previous
import functools

import jax
import jax.numpy as jnp
from jax import lax
from jax.experimental import pallas as pl
from jax.experimental.pallas import tpu as pltpu
from jax.experimental.pallas import tpu_sc as plsc

T = 4096
L = 16
NCHUNK = T // L
BIG = T + 10


def _sc_body(in_hbm, out_hbm, e_v, out_v, stage_v):
    b = lax.axis_index("s")
    f = lax.axis_index("c")

    pltpu.sync_copy(in_hbm.at[b, pl.ds(2 * f, 2), :], stage_v)

    lane = lax.iota(jnp.int32, L)
    _dnums = lax.GatherDimensionNumbers(
        offset_dims=(), collapsed_slice_dims=(0,), start_index_map=(0,)
    )
    _first = jnp.zeros((L, 1), jnp.int32)
    _last = jnp.full((L, 1), L - 1, jnp.int32)
    _shl = jnp.minimum(lane + 1, L - 1)[:, None]
    _shr = jnp.maximum(lane - 1, 0)[:, None]

    def _gat(v, idx):
        return lax.gather(
            v, idx, _dnums, (1,), mode=lax.GatherScatterMode.PROMISE_IN_BOUNDS
        )

    @plsc.parallel_loop(
        0,
        NCHUNK,
        unroll=4,
        carry=(jnp.full((L,), -BIG, jnp.int32), jnp.zeros((L,), jnp.float32)),
    )
    def _bwd(i, carry):
        cmax, nf = carry
        base = (NCHUNK - 1 - i) * L
        cur = stage_v[1, pl.ds(base, L)]
        nxt = jnp.where(lane == L - 1, nf, _gat(cur, _shl))
        on = nxt > cur
        negpos = jnp.where(on, -(base + 1) - lane, -BIG)
        comb = jnp.maximum(jnp.flip(plsc.cummax(jnp.flip(negpos, 0)), 0), cmax)
        e_v[pl.ds(base, L)] = jnp.minimum(8 - comb, T)
        return _gat(comb, _first), _gat(cur, _first)

    @plsc.parallel_loop(
        0,
        NCHUNK,
        unroll=4,
        carry=(jnp.zeros((L,), jnp.int32), jnp.zeros((L,), jnp.float32)),
    )
    def _fwd(i, carry):
        mmax, pl_last = carry
        base = i * L
        pcur = stage_v[0, pl.ds(base, L)]
        pprev = jnp.where(lane == 0, pl_last, _gat(pcur, _shr))
        e = jnp.where(pcur > pprev, e_v[pl.ds(base, L)], 0)
        comb = jnp.maximum(plsc.cummax(e), mmax)
        out_v[pl.ds(base, L)] = jnp.where(comb > base + lane, 1.0, 0.0)
        return _gat(comb, _last), _gat(pcur, _last)

    pltpu.sync_copy(out_v, out_hbm.at[b, f])


@jax.jit
def _run(x):
    mesh = plsc.VectorSubcoreMesh(core_axis_name="c", subcore_axis_name="s")
    f = pl.kernel(
        _sc_body,
        out_type=jax.ShapeDtypeStruct((16, 2, T), jnp.float32),
        mesh=mesh,
        scratch_types=[
            pltpu.VMEM((T,), jnp.int32),
            pltpu.VMEM((T,), jnp.float32),
            pltpu.VMEM((2, T), jnp.float32),
        ],
        compiler_params=pltpu.CompilerParams(
            needs_layout_passes=False, use_tc_tiling_on_sc=True
        ),
    )
    return f(x)


def kernel(gesture_labels):
    return _run(gesture_labels)
